# baseline (speedup 1.0000x reference)
import functools

import jax
import jax.numpy as jnp
from jax import lax
from jax.experimental import pallas as pl
from jax.experimental.pallas import tpu as pltpu
from jax.experimental.pallas import tpu_sc as plsc

NC = 2
NS = 16
NW = NC * NS
L = 16


def _build(M, K, NF, NP, offs):
    rpw = M // NW
    R = 64
    C = rpw // R
    n_grp = NP // L

    mesh = plsc.VectorSubcoreMesh(core_axis_name="c", subcore_axis_name="s")

    @functools.partial(
        pl.kernel,
        out_type=jax.ShapeDtypeStruct((M, NF), jnp.float32),
        mesh=mesh,
        scratch_types=[
            pltpu.VMEM((NP,), jnp.int32),
            pltpu.VMEM((R * K,), jnp.float32),
            pltpu.VMEM((R * K,), jnp.float32),
            pltpu.VMEM((R, NF), jnp.float32),
            pltpu.VMEM((R, NF), jnp.float32),
            pltpu.SemaphoreType.DMA,
            pltpu.SemaphoreType.DMA,
            pltpu.SemaphoreType.DMA,
            pltpu.SemaphoreType.DMA,
        ],
        compiler_params=pltpu.CompilerParams(
            use_tc_tiling_on_sc=False,
            needs_layout_passes=False,
            disable_bounds_checks=True,
        ),
    )
    def k(x_hbm, idx_hbm, out_hbm, idxv, xva, xvb, outva, outvb,
          isa, isb, osa, osb):
        xvs, outvs = [xva, xvb], [outva, outvb]
        isems, osems = [isa, isb], [osa, osb]
        wid = lax.axis_index("s") * NC + lax.axis_index("c")
        row0 = wid * rpw
        pltpu.sync_copy(idx_hbm, idxv)
        colgs = [idxv[pl.ds(g * L, L)] for g in range(n_grp)]

        def start_in(c):
            b = c & 1
            return pltpu.async_copy(
                x_hbm.at[pl.ds((row0 + c * R) * K, R * K)], xvs[b], isems[b]
            )

        def start_out(c):
            b = c & 1
            return pltpu.async_copy(
                outvs[b], out_hbm.at[pl.ds(row0 + c * R, R)], osems[b]
            )

        def compute(b):
            xv, outv = xvs[b], outvs[b]
            for g in range(n_grp):
                off = offs[g]

                def rowbody(r, cur):
                    vals = plsc.load_gather(xv, [cur])
                    outv[r, pl.ds(off, L)] = vals
                    return cur + K

                lax.fori_loop(0, R, rowbody, colgs[g], unroll=4)

        h_in = [None] * C
        h_out = [None] * C
        h_in[0] = start_in(0)
        for c in range(C):
            if c + 1 < C:
                h_in[c + 1] = start_in(c + 1)
            h_in[c].wait()
            if c >= 2:
                h_out[c - 2].wait()
            compute(c & 1)
            h_out[c] = start_out(c)
        h_out[C - 2].wait()
        h_out[C - 1].wait()

    return k


def kernel(x, feature_indices):
    M, K = x.shape
    NF = feature_indices.shape[0]
    G = NF // L
    rem = NF % L
    offs = [g * L for g in range(G)]
    if rem:
        idx_pad = jnp.concatenate(
            [feature_indices[: G * L], feature_indices[NF - L :]]
        )
        offs.append(NF - L)
    else:
        idx_pad = feature_indices
    k = _build(M, K, NF, idx_pad.shape[0], tuple(offs))
    return k(x.reshape(-1), idx_pad.astype(jnp.int32))


# speedup vs baseline: 1.8201x; 1.8201x over previous
import functools

import jax
import jax.numpy as jnp
from jax import lax
from jax.experimental import pallas as pl
from jax.experimental.pallas import tpu as pltpu
from jax.experimental.pallas import tpu_sc as plsc

NC = 2
NS = 16
NW = NC * NS
L = 16


def _build(M, K, NF, NP, offs):
    rpw = M // NW
    R = 64
    C = rpw // R
    n_grp = NP // L

    mesh = plsc.VectorSubcoreMesh(core_axis_name="c", subcore_axis_name="s")

    @functools.partial(
        pl.kernel,
        out_type=jax.ShapeDtypeStruct((M, NF), jnp.float32),
        mesh=mesh,
        scratch_types=[
            pltpu.VMEM((NP,), jnp.int32),
            pltpu.VMEM((R, K), jnp.float32),
            pltpu.VMEM((R, K), jnp.float32),
            pltpu.VMEM((R, NF), jnp.float32),
            pltpu.VMEM((R, NF), jnp.float32),
            pltpu.SemaphoreType.DMA,
            pltpu.SemaphoreType.DMA,
            pltpu.SemaphoreType.DMA,
            pltpu.SemaphoreType.DMA,
        ],
        compiler_params=pltpu.CompilerParams(
            use_tc_tiling_on_sc=True,
            needs_layout_passes=False,
            disable_bounds_checks=True,
        ),
    )
    def k(x_hbm, idx_hbm, out_hbm, idxv, xva, xvb, outva, outvb,
          isa, isb, osa, osb):
        xvs, outvs = [xva, xvb], [outva, outvb]
        isems, osems = [isa, isb], [osa, osb]
        wid = lax.axis_index("s") * NC + lax.axis_index("c")
        row0 = wid * rpw
        pltpu.sync_copy(idx_hbm, idxv)
        colgs = [idxv[pl.ds(g * L, L)] for g in range(n_grp)]

        def start_in(c):
            b = c & 1
            return pltpu.async_copy(
                x_hbm.at[pl.ds(row0 + c * R, R)], xvs[b], isems[b]
            )

        def start_out(c):
            b = c & 1
            return pltpu.async_copy(
                outvs[b], out_hbm.at[pl.ds(row0 + c * R, R)], osems[b]
            )

        def compute(b):
            xv, outv = xvs[b], outvs[b]

            def rowbody(r, _):
                rsplat = jnp.full((L,), r, jnp.int32)
                for g in range(n_grp):
                    vals = plsc.load_gather(xv, [rsplat, colgs[g]])
                    outv[r, pl.ds(offs[g], L)] = vals
                return 0

            lax.fori_loop(0, R, rowbody, 0, unroll=2)

        h_in = [None] * C
        h_out = [None] * C
        h_in[0] = start_in(0)
        for c in range(C):
            if c + 1 < C:
                h_in[c + 1] = start_in(c + 1)
            h_in[c].wait()
            if c >= 2:
                h_out[c - 2].wait()
            compute(c & 1)
            h_out[c] = start_out(c)
        h_out[C - 2].wait()
        h_out[C - 1].wait()

    return k


def kernel(x, feature_indices):
    M, K = x.shape
    NF = feature_indices.shape[0]
    G = NF // L
    rem = NF % L
    offs = [g * L for g in range(G)]
    if rem:
        idx_pad = jnp.concatenate(
            [feature_indices[: G * L], feature_indices[NF - L :]]
        )
        offs.append(NF - L)
    else:
        idx_pad = feature_indices
    k = _build(M, K, NF, idx_pad.shape[0], tuple(offs))
    return k(x, idx_pad.astype(jnp.int32))
